# baseline (speedup 1.0000x reference)
import jax
import jax.numpy as jnp
from jax import lax
from jax.experimental import pallas as pl
from jax.experimental.pallas import tpu as pltpu
from jax.experimental.pallas import tpu_sc as plsc

_B = 16384
_D = 32
_L = 16
_NW = 32
_BPW = _B // _NW
_CH = 128


def _body(users_hbm, books_hbm, ut_hbm, bt_hbm, wb_hbm, out_hbm,
          uidx_v, bidx_v, urows_v, brows_v, w_v, out_v, sem):
    wid = lax.axis_index("s") * 2 + lax.axis_index("c")
    base = wid * _BPW

    pltpu.sync_copy(users_hbm.at[pl.ds(base, _BPW)], uidx_v)
    pltpu.sync_copy(books_hbm.at[pl.ds(base, _BPW)], bidx_v)
    pltpu.sync_copy(wb_hbm, w_v)

    handles = []
    for c in range(_BPW // _CH):
        handles.append(pltpu.async_copy(
            ut_hbm.at[uidx_v.at[pl.ds(c * _CH, _CH)]],
            urows_v.at[pl.ds(c * _CH, _CH)], sem))
        handles.append(pltpu.async_copy(
            bt_hbm.at[bidx_v.at[pl.ds(c * _CH, _CH)]],
            brows_v.at[pl.ds(c * _CH, _CH)], sem))
    for h in handles:
        h.wait()

    wvecs = [w_v[pl.ds(k * _L, _L)] for k in range(4)]
    bias = w_v[pl.ds(2 * _D, _L)][0]
    wu = [wvecs[j // _L][j % _L] for j in range(_D)]
    wk = [wvecs[2 + j // _L][j % _L] for j in range(_D)]
    lane = lax.iota(jnp.int32, _L)

    def g_body(g, carry):
        rows = g * _L + lane
        acc = jnp.full((_L,), bias, jnp.float32)
        for j in range(_D):
            cols = jnp.full((_L,), j, jnp.int32)
            uv = plsc.load_gather(urows_v, [rows, cols])
            bv = plsc.load_gather(brows_v, [rows, cols])
            acc = acc + uv * wu[j] + bv * wk[j]
        out_v[pl.ds(g * _L, _L)] = acc
        return carry

    lax.fori_loop(0, _BPW // _L, g_body, 0)
    pltpu.sync_copy(out_v, out_hbm.at[pl.ds(base, _BPW)])


@jax.jit
def kernel(users, books, user_table, book_table, W, b):
    users = users.astype(jnp.int32)
    books = books.astype(jnp.int32)
    wb = jnp.concatenate(
        [W.reshape(-1), jnp.broadcast_to(b.reshape(-1)[0], (16,))]
    ).astype(jnp.float32)

    mesh = plsc.VectorSubcoreMesh(core_axis_name="c", subcore_axis_name="s")
    run = pl.kernel(
        _body,
        out_type=jax.ShapeDtypeStruct((_B,), jnp.float32),
        mesh=mesh,
        compiler_params=pltpu.CompilerParams(
            needs_layout_passes=False, use_tc_tiling_on_sc=False),
        scratch_types=[
            pltpu.VMEM((_BPW,), jnp.int32),
            pltpu.VMEM((_BPW,), jnp.int32),
            pltpu.VMEM((_BPW, _D), jnp.float32),
            pltpu.VMEM((_BPW, _D), jnp.float32),
            pltpu.VMEM((2 * _D + _L,), jnp.float32),
            pltpu.VMEM((_BPW,), jnp.float32),
            pltpu.SemaphoreType.DMA,
        ],
    )
    out = run(users, books, user_table, book_table, wb)
    return out.reshape(_B, 1)
